# baseline (speedup 1.0000x reference)
import jax
import jax.numpy as jnp
from jax.experimental import pallas as pl
from jax.experimental.pallas import tpu as pltpu

_ROWS = 2048


def _rowsumsq(x):
    acc = x[:, 0:128] * x[:, 0:128]
    for k in range(1, 16):
        c = x[:, k * 128:(k + 1) * 128]
        acc = acc + c * c
    b = acc[:, 0:8]
    for g in range(1, 16):
        b = b + acc[:, 8 * g:8 * g + 8]
    c4 = b[:, 0:4] + b[:, 4:8]
    c2 = c4[:, 0:2] + c4[:, 2:4]
    return c2[:, 0:1] + c2[:, 1:2]


def _gating_block(x_ref, wt_ref, g_ref, cs_ref,
                  probs_ref, logits_ref, mask_ref, gated_ref):
    x = x_ref[...]
    wt = wt_ref[...]
    g = g_ref[...]
    cs = cs_ref[...]

    wnt = wt / jnp.maximum(jnp.sqrt(cs), 1e-12)
    xn = x / jnp.maximum(jnp.sqrt(_rowsumsq(x)), 1e-12)
    sig = 1.0 / (1.0 + jnp.exp(-g))

    raw_t = jax.lax.dot_general(wnt, xn, (((1,), (1,)), ((), ())),
                                preferred_element_type=jnp.float32)
    logits = raw_t.T - sig

    gated = jnp.maximum(logits, 0.0)
    mask = (logits > 0.0).astype(jnp.float32)
    inactive = jnp.sum(mask, axis=1, keepdims=True) == 0.0

    col = jax.lax.broadcasted_iota(jnp.int32, logits.shape, 1)
    ninf = jnp.float32(-jnp.inf)
    m1 = jnp.max(logits, axis=1, keepdims=True)
    i1 = jnp.min(jnp.where(logits == m1, col, logits.shape[1]),
                 axis=1, keepdims=True)
    l2 = jnp.where(col == i1, ninf, logits)
    m2 = jnp.max(l2, axis=1, keepdims=True)
    i2 = jnp.min(jnp.where(l2 == m2, col, logits.shape[1]),
                 axis=1, keepdims=True)
    fb = (col == i1) | (col == i2)

    maskf = jnp.where(inactive & fb, 1.0, mask)
    masked = jnp.where(maskf > 0.0, gated, ninf)
    rowmax = jnp.max(masked, axis=1, keepdims=True)
    e = jnp.exp(masked - rowmax)
    probs = e / jnp.sum(e, axis=1, keepdims=True)

    probs_ref[...] = probs
    logits_ref[...] = logits
    mask_ref[...] = maskf
    gated_ref[...] = gated


def kernel(hidden_states, sim_matrix, gates):
    b, t, c = hidden_states.shape
    n = b * t
    e = sim_matrix.shape[1]
    flat = hidden_states.reshape(n, c)
    simT = sim_matrix.T
    g2 = gates.reshape(1, e)
    cs = jnp.sum(simT * simT, axis=-1, keepdims=True)

    grid = (n // _ROWS,)
    out_sds = jax.ShapeDtypeStruct((n, e), jnp.float32)
    outs = pl.pallas_call(
        _gating_block,
        grid=grid,
        in_specs=[
            pl.BlockSpec((_ROWS, c), lambda i: (i, 0)),
            pl.BlockSpec((e, c), lambda i: (0, 0)),
            pl.BlockSpec((1, e), lambda i: (0, 0)),
            pl.BlockSpec((e, 1), lambda i: (0, 0)),
        ],
        out_specs=[pl.BlockSpec((_ROWS, e), lambda i: (i, 0))] * 4,
        out_shape=[out_sds] * 4,
        compiler_params=pltpu.CompilerParams(
            dimension_semantics=("parallel",)),
    )(flat, simT, g2, cs)
    return tuple(outs)


# speedup vs baseline: 1.0229x; 1.0229x over previous
import jax
import jax.numpy as jnp
from jax.experimental import pallas as pl
from jax.experimental.pallas import tpu as pltpu

_ROWS = 1024


def _rowsumsq(x_ref):
    c0 = x_ref[:, pl.ds(0, 128)]
    acc = c0 * c0
    for k in range(1, 16):
        c = x_ref[:, pl.ds(k * 128, 128)]
        acc = acc + c * c
    b = acc[:, 0:8]
    for g in range(1, 16):
        b = b + acc[:, 8 * g:8 * g + 8]
    c4 = b[:, 0:4] + b[:, 4:8]
    c2 = c4[:, 0:2] + c4[:, 2:4]
    return c2[:, 0:1] + c2[:, 1:2]


def _gating_block(x_ref, wt_ref, g_ref, cs_ref,
                  probs_ref, logits_ref, mask_ref, gated_ref):
    wt = wt_ref[...]
    g = g_ref[...]
    cs = cs_ref[...]

    wnt = wt / jnp.maximum(jnp.sqrt(cs), 1e-12)
    xn = x_ref[...] / jnp.maximum(jnp.sqrt(_rowsumsq(x_ref)), 1e-12)
    sig = 1.0 / (1.0 + jnp.exp(-g))

    raw_t = jax.lax.dot_general(wnt, xn, (((1,), (1,)), ((), ())),
                                preferred_element_type=jnp.float32)
    logits = raw_t.T - sig

    gated = jnp.maximum(logits, 0.0)
    mask = (logits > 0.0).astype(jnp.float32)
    inactive = jnp.sum(mask, axis=1, keepdims=True) == 0.0

    col = jax.lax.broadcasted_iota(jnp.int32, logits.shape, 1)
    ninf = jnp.float32(-jnp.inf)
    m1 = jnp.max(logits, axis=1, keepdims=True)
    i1 = jnp.min(jnp.where(logits == m1, col, logits.shape[1]),
                 axis=1, keepdims=True)
    l2 = jnp.where(col == i1, ninf, logits)
    m2 = jnp.max(l2, axis=1, keepdims=True)
    i2 = jnp.min(jnp.where(l2 == m2, col, logits.shape[1]),
                 axis=1, keepdims=True)
    fb = (col == i1) | (col == i2)

    maskf = jnp.where(inactive & fb, 1.0, mask)
    masked = jnp.where(maskf > 0.0, gated, ninf)
    rowmax = jnp.max(masked, axis=1, keepdims=True)
    e = jnp.exp(masked - rowmax)
    probs = e / jnp.sum(e, axis=1, keepdims=True)

    probs_ref[...] = probs
    logits_ref[...] = logits
    mask_ref[...] = maskf
    gated_ref[...] = gated


def kernel(hidden_states, sim_matrix, gates):
    b, t, c = hidden_states.shape
    n = b * t
    e = sim_matrix.shape[1]
    flat = hidden_states.reshape(n, c)
    simT = sim_matrix.T
    g2 = gates.reshape(1, e)
    cs = jnp.sum(simT * simT, axis=-1, keepdims=True)

    grid = (n // _ROWS,)
    out_sds = jax.ShapeDtypeStruct((n, e), jnp.float32)
    outs = pl.pallas_call(
        _gating_block,
        grid=grid,
        in_specs=[
            pl.BlockSpec((_ROWS, c), lambda i: (i, 0)),
            pl.BlockSpec((e, c), lambda i: (0, 0)),
            pl.BlockSpec((1, e), lambda i: (0, 0)),
            pl.BlockSpec((e, 1), lambda i: (0, 0)),
        ],
        out_specs=[pl.BlockSpec((_ROWS, e), lambda i: (i, 0))] * 4,
        out_shape=[out_sds] * 4,
        compiler_params=pltpu.CompilerParams(
            dimension_semantics=("parallel",)),
    )(flat, simT, g2, cs)
    return tuple(outs)
